# baseline (speedup 1.0000x reference)
import functools

import jax
import jax.numpy as jnp
from jax import lax
from jax.experimental import pallas as pl
from jax.experimental.pallas import tpu as pltpu
from jax.experimental.pallas import tpu_sc as plsc

_N = 50000
_E = 1600000
_DIN = 10
_HID = 10

_NC = 2
_NS = 16
_NW = _NC * _NS
_CH = 80
_EPT = _E // _NW
_NCHUNK = _EPT // _CH
_NPAD = 50048
_EPAD = 1605632
_RPT = _NPAD // _NS
_ZCH = 184

_f32 = jnp.float32


def _silu(z):
    return z / (1.0 + jnp.exp(-z))


def _prep_nodes_body(x_ref, c_ref, w1a_ref, w1b_ref, w1c_ref, b1_ref,
                     src_ref, dst_ref):
    x = x_ref[...]
    c = c_ref[...]
    q = jnp.sum(c * c, axis=1, keepdims=True)
    w1c = w1c_ref[...]
    sa = jnp.dot(x, w1a_ref[...], preferred_element_type=_f32) + q * w1c
    sb = (jnp.dot(x, w1b_ref[...], preferred_element_type=_f32)
          + q * w1c + b1_ref[...])
    z3 = jnp.zeros_like(c)
    src_ref[...] = jnp.concatenate([sa, c, z3], axis=1)
    dst_ref[...] = jnp.concatenate([sb, c, z3], axis=1)


def _prep_nodes(x, coords, w1a, w1b, w1c, b1):
    bn = 2000
    grid = _N // bn
    full = lambda a: pl.BlockSpec(a.shape, lambda i: (0,) * a.ndim)
    return pl.pallas_call(
        _prep_nodes_body,
        grid=(grid,),
        in_specs=[
            pl.BlockSpec((bn, _DIN), lambda i: (i, 0)),
            pl.BlockSpec((bn, 3), lambda i: (i, 0)),
            full(w1a), full(w1b), full(w1c), full(b1),
        ],
        out_specs=[
            pl.BlockSpec((bn, 16), lambda i: (i, 0)),
            pl.BlockSpec((bn, 16), lambda i: (i, 0)),
        ],
        out_shape=[
            jax.ShapeDtypeStruct((_N, 16), _f32),
            jax.ShapeDtypeStruct((_N, 16), _f32),
        ],
    )(x, coords, w1a, w1b, w1c, b1)


def _prep_eft_body(ef_ref, w1d_ref, *out_refs):
    x = ef_ref[...]
    cols = [x[:, j] for j in range(4)]
    for k in range(_HID):
        acc = cols[0] * w1d_ref[0, k]
        for j in range(1, 4):
            acc = acc + cols[j] * w1d_ref[j, k]
        out_refs[k][...] = acc


def _prep_eft(edge_feat, w1d):
    be = 8192
    grid_i = _EPAD // be
    return pl.pallas_call(
        _prep_eft_body,
        grid=(grid_i,),
        in_specs=[
            pl.BlockSpec((be, 4), lambda i: (i, 0)),
            pl.BlockSpec(memory_space=pltpu.SMEM),
        ],
        out_specs=[pl.BlockSpec((be,), lambda i: (i,))] * _HID,
        out_shape=[jax.ShapeDtypeStruct((_EPAD,), _f32)] * _HID,
    )(edge_feat, w1d)


_RD = 4
_RI = 6
_RM = 2


def _edge_group(g, bo, bm, ro, srows, drows, efb, wv, mb):
    rowc = lax.iota(jnp.int32, 16) + (g * 16)
    rowi = rowc + bo

    def col(ref, k):
        return plsc.load_gather(ref, [rowi, jnp.full((16,), k, jnp.int32)])

    cs = [col(srows, 10 + a) for a in range(3)]
    cd = [col(drows, 10 + a) for a in range(3)]
    t = (cs[0] * cd[0] + cs[1] * cd[1] + cs[2] * cd[2]) * (-2.0)

    m1 = []
    for k in range(_HID):
        z = (col(srows, k) + col(drows, k) + t * wv[k, :]
             + efb[ro + k, pl.ds(g * 16, 16)])
        m1.append(_silu(z))

    z2 = [wv[110 + j, :] for j in range(_HID)]
    for k in range(_HID):
        mk = m1[k]
        for j in range(_HID):
            z2[j] = z2[j] + mk * wv[10 + k * _HID + j, :]

    for j in range(_HID):
        plsc.store_scatter(mb, [rowc + bm, jnp.full((16,), j, jnp.int32)],
                           _silu(z2[j]))


def _sc_body(srctab, dsttab, efeat, eidx, wsplat, out,
             acc, wv, ibuf, srows, drows, efb, mb, zb,
             semg, semd, seme, semi, semsc):
    c = lax.axis_index("c")
    s = lax.axis_index("s")
    wid = s * _NC + c
    e0 = wid * _EPT
    last = _NCHUNK - 1

    pltpu.sync_copy(wsplat, wv)

    zf = jnp.zeros((16,), _f32)
    for i in range(_ZCH):
        zb[i, :] = zf
    for i in range(_RM * _CH):
        mb[i, :] = zf
    r0 = s * _RPT
    for k in range(_RPT // _ZCH):
        pltpu.sync_copy(zb, acc.at[pl.ds(r0 + k * _ZCH, _ZCH), :])
    plsc.subcore_barrier()

    def idx_copy(chunk_id, slot):
        return pltpu.make_async_copy(
            eidx.at[:, pl.ds(e0 + chunk_id * _CH, _CH)],
            ibuf.at[pl.ds(2 * slot, 2)],
            semi.at[slot])

    def gathers(chunk_id, slot_r, slot_b):
        bo = slot_b * _CH
        return (
            pltpu.make_async_copy(srctab.at[ibuf.at[2 * slot_r]],
                                  srows.at[pl.ds(bo, _CH)], semg.at[slot_b]),
            pltpu.make_async_copy(dsttab.at[ibuf.at[2 * slot_r + 1]],
                                  drows.at[pl.ds(bo, _CH)], semd.at[slot_b]),
            pltpu.make_async_copy(
                efeat.at[:, pl.ds(e0 + chunk_id * _CH, _CH)],
                efb.at[pl.ds(slot_b * _HID, _HID)], seme.at[slot_b]),
        )

    def scatter(i_, slot_m, slot_r):
        return pltpu.make_async_copy(mb.at[pl.ds(slot_m * _CH, _CH)],
                                     acc.at[ibuf.at[2 * slot_r + 1]],
                                     semsc.at[slot_m])

    for k in range(3):
        ck = jnp.int32(min(k, last))
        idx_copy(ck, k).start()
        idx_copy(ck, k).wait()
    idx_copy(jnp.int32(min(3, last)), 3).start()
    for k in range(3):
        for cp in gathers(jnp.int32(min(k, last)), k, k):
            cp.start()

    def step(i, carry):
        b0 = lax.rem(i, _RD)
        b3 = lax.rem(i + 3, _RD)
        ri0 = lax.rem(i, _RI)
        ri3 = lax.rem(i + 3, _RI)
        ri4 = lax.rem(i + 4, _RI)
        bm = lax.rem(i, _RM)
        c3 = jnp.minimum(i + 3, last)
        c4 = jnp.minimum(i + 4, last)

        idx_copy(c3, ri3).wait()
        for cp in gathers(c3, ri3, b3):
            cp.start()
        idx_copy(c4, ri4).start()

        for cp in gathers(i, ri0, b0):
            cp.wait()
        bo = b0 * _CH
        bmo = bm * _CH
        ro = b0 * _HID
        for g in range(_CH // 16):
            _edge_group(g, bo, bmo, ro, srows, drows, efb, wv, mb)

        @pl.when(i > 0)
        def _():
            scatter(i - 1, lax.rem(i + 1, _RM), lax.rem(i + 5, _RI)).wait()
        scatter(i, bm, ri0).start(add=True)
        return carry

    lax.fori_loop(0, _NCHUNK, step, 0)

    for t in range(3):
        ii = _NCHUNK - 3 + t
        for cp in gathers(jnp.int32(last), (ii + 3) % _RI, (ii + 3) % _RD):
            cp.wait()
    idx_copy(jnp.int32(last), (_NCHUNK + 3) % _RI).wait()
    scatter(_NCHUNK - 1, (_NCHUNK - 1) % _RM, (_NCHUNK - 1) % _RI).wait()

    plsc.subcore_barrier()

    for k in range(_RPT // _ZCH):
        rr = r0 + k * _ZCH
        pltpu.sync_copy(acc.at[pl.ds(rr, _ZCH), :], zb)
        pltpu.sync_copy(zb, out.at[c, pl.ds(rr, _ZCH), :])


_sc_edges = pl.kernel(
    _sc_body,
    out_type=jax.ShapeDtypeStruct((_NC, _NPAD, 16), _f32),
    mesh=plsc.VectorSubcoreMesh(core_axis_name="c", subcore_axis_name="s",
                                num_cores=_NC, num_subcores=_NS),
    compiler_params=pltpu.CompilerParams(needs_layout_passes=False,
                                         use_tc_tiling_on_sc=False),
    scratch_types=[
        pltpu.VMEM_SHARED((_NPAD, 16), _f32),
        pltpu.VMEM((120, 16), _f32),
        pltpu.VMEM((2 * _RI, _CH), jnp.int32),
        pltpu.VMEM((_RD * _CH, 16), _f32),
        pltpu.VMEM((_RD * _CH, 16), _f32),
        pltpu.VMEM((_RD * _HID, _CH), _f32),
        pltpu.VMEM((_RM * _CH, 16), _f32),
        pltpu.VMEM((_ZCH, 16), _f32),
        pltpu.SemaphoreType.DMA((_RD,)),
        pltpu.SemaphoreType.DMA((_RD,)),
        pltpu.SemaphoreType.DMA((_RD,)),
        pltpu.SemaphoreType.DMA((_RI,)),
        pltpu.SemaphoreType.DMA((_RM,)),
    ],
)


def _post_body(x_ref, p_ref, nw1a_ref, nw1b_ref, nb1_ref, nw2_ref, nb2_ref,
               fcw_ref, fcb_ref, fc2w_ref, fc2b_ref, o_ref):
    x = x_ref[...]
    hn = p_ref[0, :, :_HID] + p_ref[1, :, :_HID]
    a = (jnp.dot(x, nw1a_ref[...], preferred_element_type=_f32)
         + jnp.dot(hn, nw1b_ref[...], preferred_element_type=_f32)
         + nb1_ref[...])
    h = jnp.dot(_silu(a), nw2_ref[...], preferred_element_type=_f32) + nb2_ref[...]
    r = jnp.maximum(h, 0.0)
    o = jnp.dot(r, fcw_ref[...], preferred_element_type=_f32) + fcb_ref[...]
    o = jnp.dot(o, fc2w_ref[...], preferred_element_type=_f32) + fc2b_ref[...]
    mx = jnp.max(o, axis=1, keepdims=True)
    e = jnp.exp(o - mx)
    o_ref[...] = e / jnp.sum(e, axis=1, keepdims=True)


def _post(x, partials, nw1a, nw1b, nb1, nw2, nb2, fcw, fcb, fc2w, fc2b):
    bn = 2000
    grid = _N // bn
    full = lambda a: pl.BlockSpec(a.shape, lambda i: (0,) * a.ndim)
    return pl.pallas_call(
        _post_body,
        grid=(grid,),
        in_specs=[
            pl.BlockSpec((bn, _DIN), lambda i: (i, 0)),
            pl.BlockSpec((_NC, bn, 16), lambda i: (0, i, 0)),
            full(nw1a), full(nw1b), full(nb1), full(nw2), full(nb2),
            full(fcw), full(fcb), full(fc2w), full(fc2b),
        ],
        out_specs=pl.BlockSpec((bn, 3), lambda i: (i, 0)),
        out_shape=jax.ShapeDtypeStruct((_N, 3), _f32),
    )(x, partials, nw1a, nw1b, nb1, nw2, nb2, fcw, fcb, fc2w, fc2b)


def kernel(edge_index, x, coords, edge_feat, params):
    p = params
    w1 = p["edge_w1"]
    w1a, w1b, w1c, w1d = (w1[:_DIN], w1[_DIN:2 * _DIN], w1[2 * _DIN],
                          w1[2 * _DIN + 1:])
    b1 = p["edge_b1"]

    src_tab, dst_tab = _prep_nodes(x, coords, w1a, w1b,
                                   w1c.reshape(1, _HID), b1.reshape(1, _HID))
    eft = jnp.concatenate(_prep_eft(edge_feat, w1d)).reshape(_HID, _EPAD)

    wvec = jnp.concatenate([w1c, p["edge_w2"].reshape(-1), p["edge_b2"]])
    wsplat = jnp.broadcast_to(wvec[:, None], (120, 16)).astype(_f32)

    partials = _sc_edges(src_tab, dst_tab, eft, edge_index, wsplat)

    return _post(x, partials,
                 p["node_w1"][:_DIN], p["node_w1"][_DIN:],
                 p["node_b1"].reshape(1, _HID),
                 p["node_w2"], p["node_b2"].reshape(1, _DIN),
                 p["fc_w"], p["fc_b"].reshape(1, _DIN),
                 p["fc2_w"], p["fc2_b"].reshape(1, 3))


# speedup vs baseline: 8.6393x; 8.6393x over previous
import functools

import jax
import jax.numpy as jnp
from jax import lax
from jax.experimental import pallas as pl
from jax.experimental.pallas import tpu as pltpu
from jax.experimental.pallas import tpu_sc as plsc

_N = 50000
_E = 1600000
_DIN = 10
_HID = 10

_NC = 2
_NS = 16
_NW = _NC * _NS
_CH = 80
_EPT = _E // _NW
_NCHUNK = _EPT // _CH
_NPAD = 50048
_EPAD = 1605632
_RPT = _NPAD // _NS
_ZCH = 184

_f32 = jnp.float32


def _silu(z):
    return z / (1.0 + jnp.exp(-z))


def _prep_nodes_body(x_ref, c_ref, w1a_ref, w1b_ref, w1c_ref, b1_ref,
                     src_ref, dst_ref):
    x = x_ref[...]
    c = c_ref[...]
    q = jnp.sum(c * c, axis=1, keepdims=True)
    w1c = w1c_ref[...]
    sa = jnp.dot(x, w1a_ref[...], preferred_element_type=_f32) + q * w1c
    sb = (jnp.dot(x, w1b_ref[...], preferred_element_type=_f32)
          + q * w1c + b1_ref[...])
    z3 = jnp.zeros_like(c)
    src_ref[...] = jnp.concatenate([sa, c, z3], axis=1)
    dst_ref[...] = jnp.concatenate([sb, c, z3], axis=1)


def _prep_nodes(x, coords, w1a, w1b, w1c, b1):
    bn = 2000
    grid = _N // bn
    full = lambda a: pl.BlockSpec(a.shape, lambda i: (0,) * a.ndim)
    return pl.pallas_call(
        _prep_nodes_body,
        grid=(grid,),
        in_specs=[
            pl.BlockSpec((bn, _DIN), lambda i: (i, 0)),
            pl.BlockSpec((bn, 3), lambda i: (i, 0)),
            full(w1a), full(w1b), full(w1c), full(b1),
        ],
        out_specs=[
            pl.BlockSpec((bn, 16), lambda i: (i, 0)),
            pl.BlockSpec((bn, 16), lambda i: (i, 0)),
        ],
        out_shape=[
            jax.ShapeDtypeStruct((_N, 16), _f32),
            jax.ShapeDtypeStruct((_N, 16), _f32),
        ],
    )(x, coords, w1a, w1b, w1c, b1)


def _prep_eft_body(ef_ref, w1d_ref, *out_refs):
    p = jnp.dot(ef_ref[...], w1d_ref[...], preferred_element_type=_f32)
    pt = p.T
    for k in range(_HID):
        out_refs[k][...] = pt[k, :]


def _prep_eft(edge_feat, w1d):
    be = 8192
    grid_i = _EPAD // be
    return pl.pallas_call(
        _prep_eft_body,
        grid=(grid_i,),
        in_specs=[
            pl.BlockSpec((be, 4), lambda i: (i, 0)),
            pl.BlockSpec((4, _HID), lambda i: (0, 0)),
        ],
        out_specs=[pl.BlockSpec((be,), lambda i: (i,))] * _HID,
        out_shape=[jax.ShapeDtypeStruct((_EPAD,), _f32)] * _HID,
    )(edge_feat, w1d)


_RD = 4
_RI = 6
_RM = 2


def _edge_group(g, bo, bm, ro, srows, drows, efb, wv, mb):
    rowc = lax.iota(jnp.int32, 16) + (g * 16)
    rowi = rowc + bo

    def col(ref, k):
        return plsc.load_gather(ref, [rowi, jnp.full((16,), k, jnp.int32)])

    cs = [col(srows, 10 + a) for a in range(3)]
    cd = [col(drows, 10 + a) for a in range(3)]
    t = (cs[0] * cd[0] + cs[1] * cd[1] + cs[2] * cd[2]) * (-2.0)

    m1 = []
    for k in range(_HID):
        z = (col(srows, k) + col(drows, k) + t * wv[k, :]
             + efb[ro + k, pl.ds(g * 16, 16)])
        m1.append(_silu(z))

    z2 = [wv[110 + j, :] for j in range(_HID)]
    for k in range(_HID):
        mk = m1[k]
        for j in range(_HID):
            z2[j] = z2[j] + mk * wv[10 + k * _HID + j, :]

    for j in range(_HID):
        plsc.store_scatter(mb, [rowc + bm, jnp.full((16,), j, jnp.int32)],
                           _silu(z2[j]))


def _sc_body(srctab, dsttab, efeat, eidx, wsplat, out,
             acc, wv, ibuf, srows, drows, efb, mb, zb,
             semg, semd, seme, semi, semsc):
    c = lax.axis_index("c")
    s = lax.axis_index("s")
    wid = s * _NC + c
    e0 = wid * _EPT
    last = _NCHUNK - 1

    pltpu.sync_copy(wsplat, wv)

    zf = jnp.zeros((16,), _f32)
    for i in range(_ZCH):
        zb[i, :] = zf
    for i in range(_RM * _CH):
        mb[i, :] = zf
    r0 = s * _RPT
    for k in range(_RPT // _ZCH):
        pltpu.sync_copy(zb, acc.at[pl.ds(r0 + k * _ZCH, _ZCH), :])
    plsc.subcore_barrier()

    def idx_copy(chunk_id, slot):
        return pltpu.make_async_copy(
            eidx.at[:, pl.ds(e0 + chunk_id * _CH, _CH)],
            ibuf.at[pl.ds(2 * slot, 2)],
            semi.at[slot])

    def gathers(chunk_id, slot_r, slot_b):
        bo = slot_b * _CH
        return (
            pltpu.make_async_copy(srctab.at[ibuf.at[2 * slot_r]],
                                  srows.at[pl.ds(bo, _CH)], semg.at[slot_b]),
            pltpu.make_async_copy(dsttab.at[ibuf.at[2 * slot_r + 1]],
                                  drows.at[pl.ds(bo, _CH)], semd.at[slot_b]),
            pltpu.make_async_copy(
                efeat.at[:, pl.ds(e0 + chunk_id * _CH, _CH)],
                efb.at[pl.ds(slot_b * _HID, _HID)], seme.at[slot_b]),
        )

    def scatter(i_, slot_m, slot_r):
        return pltpu.make_async_copy(mb.at[pl.ds(slot_m * _CH, _CH)],
                                     acc.at[ibuf.at[2 * slot_r + 1]],
                                     semsc.at[slot_m])

    for k in range(3):
        ck = jnp.int32(min(k, last))
        idx_copy(ck, k).start()
        idx_copy(ck, k).wait()
    idx_copy(jnp.int32(min(3, last)), 3).start()
    for k in range(3):
        for cp in gathers(jnp.int32(min(k, last)), k, k):
            cp.start()

    def step(i, carry):
        b0 = lax.rem(i, _RD)
        b3 = lax.rem(i + 3, _RD)
        ri0 = lax.rem(i, _RI)
        ri3 = lax.rem(i + 3, _RI)
        ri4 = lax.rem(i + 4, _RI)
        bm = lax.rem(i, _RM)
        c3 = jnp.minimum(i + 3, last)
        c4 = jnp.minimum(i + 4, last)

        idx_copy(c3, ri3).wait()
        for cp in gathers(c3, ri3, b3):
            cp.start()
        idx_copy(c4, ri4).start()

        for cp in gathers(i, ri0, b0):
            cp.wait()
        bo = b0 * _CH
        bmo = bm * _CH
        ro = b0 * _HID
        for g in range(_CH // 16):
            _edge_group(g, bo, bmo, ro, srows, drows, efb, wv, mb)

        @pl.when(i > 0)
        def _():
            scatter(i - 1, lax.rem(i + 1, _RM), lax.rem(i + 5, _RI)).wait()
        scatter(i, bm, ri0).start(add=True)
        return carry

    lax.fori_loop(0, _NCHUNK, step, 0)

    for t in range(3):
        ii = _NCHUNK - 3 + t
        for cp in gathers(jnp.int32(last), (ii + 3) % _RI, (ii + 3) % _RD):
            cp.wait()
    idx_copy(jnp.int32(last), (_NCHUNK + 3) % _RI).wait()
    scatter(_NCHUNK - 1, (_NCHUNK - 1) % _RM, (_NCHUNK - 1) % _RI).wait()

    plsc.subcore_barrier()

    for k in range(_RPT // _ZCH):
        rr = r0 + k * _ZCH
        pltpu.sync_copy(acc.at[pl.ds(rr, _ZCH), :], zb)
        pltpu.sync_copy(zb, out.at[c, pl.ds(rr, _ZCH), :])


_sc_edges = pl.kernel(
    _sc_body,
    out_type=jax.ShapeDtypeStruct((_NC, _NPAD, 16), _f32),
    mesh=plsc.VectorSubcoreMesh(core_axis_name="c", subcore_axis_name="s",
                                num_cores=_NC, num_subcores=_NS),
    compiler_params=pltpu.CompilerParams(needs_layout_passes=False,
                                         use_tc_tiling_on_sc=False),
    scratch_types=[
        pltpu.VMEM_SHARED((_NPAD, 16), _f32),
        pltpu.VMEM((120, 16), _f32),
        pltpu.VMEM((2 * _RI, _CH), jnp.int32),
        pltpu.VMEM((_RD * _CH, 16), _f32),
        pltpu.VMEM((_RD * _CH, 16), _f32),
        pltpu.VMEM((_RD * _HID, _CH), _f32),
        pltpu.VMEM((_RM * _CH, 16), _f32),
        pltpu.VMEM((_ZCH, 16), _f32),
        pltpu.SemaphoreType.DMA((_RD,)),
        pltpu.SemaphoreType.DMA((_RD,)),
        pltpu.SemaphoreType.DMA((_RD,)),
        pltpu.SemaphoreType.DMA((_RI,)),
        pltpu.SemaphoreType.DMA((_RM,)),
    ],
)


def _post_body(x_ref, p_ref, nw1a_ref, nw1b_ref, nb1_ref, nw2_ref, nb2_ref,
               fcw_ref, fcb_ref, fc2w_ref, fc2b_ref, o_ref):
    x = x_ref[...]
    hn = p_ref[0, :, :_HID] + p_ref[1, :, :_HID]
    a = (jnp.dot(x, nw1a_ref[...], preferred_element_type=_f32)
         + jnp.dot(hn, nw1b_ref[...], preferred_element_type=_f32)
         + nb1_ref[...])
    h = jnp.dot(_silu(a), nw2_ref[...], preferred_element_type=_f32) + nb2_ref[...]
    r = jnp.maximum(h, 0.0)
    o = jnp.dot(r, fcw_ref[...], preferred_element_type=_f32) + fcb_ref[...]
    o = jnp.dot(o, fc2w_ref[...], preferred_element_type=_f32) + fc2b_ref[...]
    mx = jnp.max(o, axis=1, keepdims=True)
    e = jnp.exp(o - mx)
    o_ref[...] = e / jnp.sum(e, axis=1, keepdims=True)


def _post(x, partials, nw1a, nw1b, nb1, nw2, nb2, fcw, fcb, fc2w, fc2b):
    bn = 2000
    grid = _N // bn
    full = lambda a: pl.BlockSpec(a.shape, lambda i: (0,) * a.ndim)
    return pl.pallas_call(
        _post_body,
        grid=(grid,),
        in_specs=[
            pl.BlockSpec((bn, _DIN), lambda i: (i, 0)),
            pl.BlockSpec((_NC, bn, 16), lambda i: (0, i, 0)),
            full(nw1a), full(nw1b), full(nb1), full(nw2), full(nb2),
            full(fcw), full(fcb), full(fc2w), full(fc2b),
        ],
        out_specs=pl.BlockSpec((bn, 3), lambda i: (i, 0)),
        out_shape=jax.ShapeDtypeStruct((_N, 3), _f32),
    )(x, partials, nw1a, nw1b, nb1, nw2, nb2, fcw, fcb, fc2w, fc2b)


def kernel(edge_index, x, coords, edge_feat, params):
    p = params
    w1 = p["edge_w1"]
    w1a, w1b, w1c, w1d = (w1[:_DIN], w1[_DIN:2 * _DIN], w1[2 * _DIN],
                          w1[2 * _DIN + 1:])
    b1 = p["edge_b1"]

    src_tab, dst_tab = _prep_nodes(x, coords, w1a, w1b,
                                   w1c.reshape(1, _HID), b1.reshape(1, _HID))
    eft = jnp.concatenate(_prep_eft(edge_feat, w1d)).reshape(_HID, _EPAD)

    wvec = jnp.concatenate([w1c, p["edge_w2"].reshape(-1), p["edge_b2"]])
    wsplat = jnp.broadcast_to(wvec[:, None], (120, 16)).astype(_f32)

    partials = _sc_edges(src_tab, dst_tab, eft, edge_index, wsplat)

    return _post(x, partials,
                 p["node_w1"][:_DIN], p["node_w1"][_DIN:],
                 p["node_b1"].reshape(1, _HID),
                 p["node_w2"], p["node_b2"].reshape(1, _DIN),
                 p["fc_w"], p["fc_b"].reshape(1, _DIN),
                 p["fc2_w"], p["fc2_b"].reshape(1, 3))
